# baseline (speedup 1.0000x reference)
import functools

import jax
import jax.numpy as jnp
from jax import lax
from jax.experimental import pallas as pl
from jax.experimental.pallas import tpu as pltpu
from jax.experimental.pallas import tpu_sc as plsc

_N = 10000
_E = 160000
_D = 256
_H = 256
_HH = _H // 2

_NC = 2
_NS = 16
_L = 16
_NW = _NC * _NS

_ET = _E + _N
_GP = -(-_ET // (_NW * _L))
_EPW = _GP * _L
_EP = _NW * _EPW

_RQ = 624
_RQL = _N - _RQ * (_NS - 1)


def _deg_partials_body(dst_hbm, w_hbm, out_hbm, dstv, wv, deg):
    c = lax.axis_index("c")
    s = lax.axis_index("s")
    wid = s * _NC + c
    pltpu.sync_copy(dst_hbm.at[wid, 0], dstv)
    pltpu.sync_copy(w_hbm.at[wid, 0], wv)
    zero = jnp.zeros((_L,), jnp.float32)

    def _zero(i, _):
        deg[pl.ds(i * _L, _L)] = zero
        return 0

    lax.fori_loop(0, _N // _L, _zero, 0)

    def _acc(i, _):
        d16 = dstv[pl.ds(i * _L, _L)]
        w16 = wv[pl.ds(i * _L, _L)]
        plsc.addupdate_scatter(deg, [d16], w16)
        return 0

    lax.fori_loop(0, _GP, _acc, 0)
    pltpu.sync_copy(deg, out_hbm.at[wid, 0])


def _dis_body(p_ref, dis_ref):
    deg = jnp.sum(p_ref[...], axis=0, keepdims=True)
    dis_ref[...] = jnp.where(
        deg > 0.0, lax.rsqrt(jnp.maximum(deg, 1e-12)), 0.0)


_dis_tc = pl.pallas_call(
    _dis_body,
    out_shape=jax.ShapeDtypeStruct((1, _N), jnp.float32),
)


def _edge_norm_body(dis_hbm, src_hbm, dst_hbm, w_hbm, out_hbm,
                    disv, srcv, dstv, wv, nv):
    c = lax.axis_index("c")
    s = lax.axis_index("s")
    wid = s * _NC + c
    pltpu.sync_copy(dis_hbm, disv)
    pltpu.sync_copy(src_hbm.at[wid, 0], srcv)
    pltpu.sync_copy(dst_hbm.at[wid, 0], dstv)
    pltpu.sync_copy(w_hbm.at[wid, 0], wv)

    def _grp(i, _):
        sl = pl.ds(i * _L, _L)
        a = plsc.load_gather(disv, [srcv[sl]])
        b = plsc.load_gather(disv, [dstv[sl]])
        nv[sl] = a * wv[sl] * b
        return 0

    lax.fori_loop(0, _GP, _grp, 0)
    pltpu.sync_copy(nv, out_hbm.at[wid, 0])


_NB = 6
_PF = 3
_NG = 2 * _GP


def _agg_body(t0_hbm, t1_hbm, src_hbm, dst_hbm, nrm_hbm, agg0_hbm, agg1_hbm,
              srcv, dstv, nrmv, rows, acc, gsem, ssem):
    c = lax.axis_index("c")
    s = lax.axis_index("s")

    z16 = jnp.zeros((_L,), jnp.float32)
    for i in range(_L):
        for k in range(_HH // _L):
            rows[0, i, pl.ds(k * _L, _L)] = z16
    r0 = s * _RQ
    for q in range(_RQL // _L):
        pltpu.sync_copy(rows.at[0], acc.at[pl.ds(r0 + q * _L, _L)])
    plsc.subcore_barrier()

    pltpu.sync_copy(src_hbm.at[s * 2, 0], srcv.at[pl.ds(0, _EPW)])
    pltpu.sync_copy(src_hbm.at[s * 2 + 1, 0], srcv.at[pl.ds(_EPW, _EPW)])
    pltpu.sync_copy(dst_hbm.at[s * 2, 0], dstv.at[pl.ds(0, _EPW)])
    pltpu.sync_copy(dst_hbm.at[s * 2 + 1, 0], dstv.at[pl.ds(_EPW, _EPW)])
    pltpu.sync_copy(nrm_hbm.at[s * 2, 0], nrmv.at[pl.ds(0, _EPW)])
    pltpu.sync_copy(nrm_hbm.at[s * 2 + 1, 0], nrmv.at[pl.ds(_EPW, _EPW)])

    t_hbm = t0_hbm

    def _gather(g, j):
        s16 = srcv[pl.ds(g * _L, _L)]

        @pl.when(c == 0)
        def _():
            pltpu.async_copy(t0_hbm.at[s16], rows.at[j], gsem)

        @pl.when(c == 1)
        def _():
            pltpu.async_copy(t1_hbm.at[s16], rows.at[j], gsem)

    def _wait_gather(j):
        z = jnp.zeros((_L,), jnp.int32)
        pltpu.make_async_copy(t_hbm.at[z], rows.at[j], gsem).wait()

    def _scale_scatter(g, j):
        base = g * _L
        for i2 in range(_L):
            eidx = (jnp.zeros((_L,), jnp.int32) + base + i2)
            spl = plsc.load_gather(nrmv, [eidx])
            for k in range(_HH // _L):
                ksl = pl.ds(k * _L, _L)
                rows[j, i2, ksl] = rows[j, i2, ksl] * spl
        d16 = dstv[pl.ds(g * _L, _L)]
        pltpu.async_copy(rows.at[j], acc.at[d16], ssem, add=True)

    def _drain_scatter(j):
        z = jnp.zeros((_L,), jnp.int32)
        pltpu.make_async_copy(rows.at[j], acc.at[z], ssem).wait()

    for j in range(_PF):
        _gather(j, j)

    def _chunk(m, _):
        g0 = m * _NB
        for jj in range(_NB):
            g = g0 + jj

            @pl.when(g >= _PF)
            def _():
                _drain_scatter((jj + _PF) % _NB)

            @pl.when(g + _PF < _NG)
            def _():
                _gather(g + _PF, (jj + _PF) % _NB)

            _wait_gather(jj)
            _scale_scatter(g, jj)
        return 0

    lax.fori_loop(0, _NG // _NB, _chunk, 0)

    for jj in range(_PF):
        _drain_scatter(jj)

    plsc.subcore_barrier()

    @pl.when(c == 0)
    def _():
        pltpu.sync_copy(acc.at[pl.ds(s * _RQ, _RQ)],
                        agg0_hbm.at[pl.ds(s * _RQ, _RQ)])

    @pl.when(c == 1)
    def _():
        pltpu.sync_copy(acc.at[pl.ds(s * _RQ, _RQ)],
                        agg1_hbm.at[pl.ds(s * _RQ, _RQ)])

    tail = _RQ * (_NS - 1)

    @pl.when(jnp.logical_and(c == 0, s == _NS - 1))
    def _():
        pltpu.sync_copy(acc.at[pl.ds(tail + _RQ, _RQL - _RQ)],
                        agg0_hbm.at[pl.ds(tail + _RQ, _RQL - _RQ)])

    @pl.when(jnp.logical_and(c == 1, s == _NS - 1))
    def _():
        pltpu.sync_copy(acc.at[pl.ds(tail + _RQ, _RQL - _RQ)],
                        agg1_hbm.at[pl.ds(tail + _RQ, _RQL - _RQ)])


@functools.cache
def _sc_kernels():
    mesh = plsc.VectorSubcoreMesh(core_axis_name="c", subcore_axis_name="s",
                                  num_cores=_NC, num_subcores=_NS)
    params = pltpu.CompilerParams(needs_layout_passes=False)
    deg = pl.kernel(
        _deg_partials_body,
        out_type=jax.ShapeDtypeStruct((_NW, 1, _N), jnp.float32),
        mesh=mesh,
        compiler_params=params,
        scratch_types=[
            pltpu.VMEM((_EPW,), jnp.int32),
            pltpu.VMEM((_EPW,), jnp.float32),
            pltpu.VMEM((_N,), jnp.float32),
        ],
    )
    enorm = pl.kernel(
        _edge_norm_body,
        out_type=jax.ShapeDtypeStruct((_NW, 1, _EPW), jnp.float32),
        mesh=mesh,
        compiler_params=params,
        scratch_types=[
            pltpu.VMEM((_N,), jnp.float32),
            pltpu.VMEM((_EPW,), jnp.int32),
            pltpu.VMEM((_EPW,), jnp.int32),
            pltpu.VMEM((_EPW,), jnp.float32),
            pltpu.VMEM((_EPW,), jnp.float32),
        ],
    )
    agg = pl.kernel(
        _agg_body,
        out_type=[jax.ShapeDtypeStruct((_N, _HH), jnp.float32)] * 2,
        mesh=mesh,
        compiler_params=params,
        scratch_types=[
            pltpu.VMEM((2 * _EPW,), jnp.int32),
            pltpu.VMEM((2 * _EPW,), jnp.int32),
            pltpu.VMEM((2 * _EPW,), jnp.float32),
            pltpu.VMEM((_NB, _L, _HH), jnp.float32),
            pltpu.VMEM_SHARED((_N, _HH), jnp.float32),
            pltpu.SemaphoreType.DMA,
            pltpu.SemaphoreType.DMA,
        ],
    )
    return deg, enorm, agg


_RB = 2000


def _mm0_body(x_ref, w_ref, t0_ref, t1_ref):
    t = lax.dot_general(x_ref[...], w_ref[...], (((1,), (1,)), ((), ())),
                        preferred_element_type=jnp.float32)
    t0_ref[...] = t[:, :_HH]
    t1_ref[...] = t[:, _HH:]


_mm0 = pl.pallas_call(
    _mm0_body,
    grid=(_N // _RB,),
    in_specs=[
        pl.BlockSpec((_RB, _D), lambda i: (i, 0)),
        pl.BlockSpec((_H, _D), lambda i: (0, 0)),
    ],
    out_specs=[pl.BlockSpec((_RB, _HH), lambda i: (i, 0))] * 2,
    out_shape=[jax.ShapeDtypeStruct((_N, _HH), jnp.float32)] * 2,
)


def _cm_body(a0_ref, a1_ref, b_ref, w_ref, t0_ref, t1_ref):
    b = b_ref[...]
    h0 = jnp.maximum(a0_ref[...] + b[:, :_HH], 0.0)
    h1 = jnp.maximum(a1_ref[...] + b[:, _HH:], 0.0)
    w = w_ref[...]
    t = (lax.dot_general(h0, w[:, :_HH], (((1,), (1,)), ((), ())),
                         preferred_element_type=jnp.float32)
         + lax.dot_general(h1, w[:, _HH:], (((1,), (1,)), ((), ())),
                           preferred_element_type=jnp.float32))
    t0_ref[...] = t[:, :_HH]
    t1_ref[...] = t[:, _HH:]


_cmm = pl.pallas_call(
    _cm_body,
    grid=(_N // _RB,),
    in_specs=[
        pl.BlockSpec((_RB, _HH), lambda i: (i, 0)),
        pl.BlockSpec((_RB, _HH), lambda i: (i, 0)),
        pl.BlockSpec((1, _H), lambda i: (0, 0)),
        pl.BlockSpec((_H, _H), lambda i: (0, 0)),
    ],
    out_specs=[pl.BlockSpec((_RB, _HH), lambda i: (i, 0))] * 2,
    out_shape=[jax.ShapeDtypeStruct((_N, _HH), jnp.float32)] * 2,
)


def _fin_body(a0_ref, a1_ref, b_ref, o_ref):
    b = b_ref[...]
    o_ref[:, :_HH] = a0_ref[...] + b[:, :_HH]
    o_ref[:, _HH:] = a1_ref[...] + b[:, _HH:]


_fin = pl.pallas_call(
    _fin_body,
    grid=(_N // _RB,),
    in_specs=[
        pl.BlockSpec((_RB, _HH), lambda i: (i, 0)),
        pl.BlockSpec((_RB, _HH), lambda i: (i, 0)),
        pl.BlockSpec((1, _H), lambda i: (0, 0)),
    ],
    out_specs=pl.BlockSpec((_RB, _H), lambda i: (i, 0)),
    out_shape=jax.ShapeDtypeStruct((_N, _H), jnp.float32),
)


def kernel(x, edge_index, edge_weight, batch,
           W0, b0, W1, b1, W2, b2, W3, b3, W4, b4):
    del batch
    loop = jnp.arange(_N, dtype=jnp.int32)
    padi = jnp.zeros((_EP - _ET,), jnp.int32)
    src_p = jnp.concatenate([edge_index[0], loop, padi]).reshape(_NW, 1, _EPW)
    dst_p = jnp.concatenate([edge_index[1], loop, padi]).reshape(_NW, 1, _EPW)
    w_p = jnp.concatenate([
        edge_weight.astype(jnp.float32),
        jnp.ones((_N,), jnp.float32),
        jnp.zeros((_EP - _ET,), jnp.float32),
    ]).reshape(_NW, 1, _EPW)

    _deg_partials, _edge_norm, _agg = _sc_kernels()
    partials = _deg_partials(dst_p, w_p)
    dis = _dis_tc(partials.reshape(_NW, _N)).reshape(_N)
    nrm = _edge_norm(dis, src_p, dst_p, w_p)

    t0, t1 = _mm0(x, W0)
    a0, a1 = _agg(t0, t1, src_p, dst_p, nrm)
    for (bb, WW) in ((b0, W1), (b1, W2), (b2, W3), (b3, W4)):
        t0, t1 = _cmm(a0, a1, bb.reshape(1, _H), WW)
        a0, a1 = _agg(t0, t1, src_p, dst_p, nrm)
    return _fin(a0, a1, b4.reshape(1, _H))


# speedup vs baseline: 1.2040x; 1.2040x over previous
import functools

import jax
import jax.numpy as jnp
from jax import lax
from jax.experimental import pallas as pl
from jax.experimental.pallas import tpu as pltpu
from jax.experimental.pallas import tpu_sc as plsc

_N = 10000
_E = 160000
_D = 256
_H = 256
_HH = _H // 2

_NC = 2
_NS = 16
_L = 16
_NW = _NC * _NS

_ET = _E + _N
_GP = -(-_ET // (_NW * _L))
_EPW = _GP * _L
_EP = _NW * _EPW

_RQ = 624
_RQL = _N - _RQ * (_NS - 1)


def _deg_partials_body(dst_hbm, w_hbm, out_hbm, dstv, wv, deg):
    c = lax.axis_index("c")
    s = lax.axis_index("s")
    wid = s * _NC + c
    pltpu.sync_copy(dst_hbm.at[wid, 0], dstv)
    pltpu.sync_copy(w_hbm.at[wid, 0], wv)
    zero = jnp.zeros((_L,), jnp.float32)

    def _zero(i, _):
        deg[pl.ds(i * _L, _L)] = zero
        return 0

    lax.fori_loop(0, _N // _L, _zero, 0)

    def _acc(i, _):
        d16 = dstv[pl.ds(i * _L, _L)]
        w16 = wv[pl.ds(i * _L, _L)]
        plsc.addupdate_scatter(deg, [d16], w16)
        return 0

    lax.fori_loop(0, _GP, _acc, 0)
    pltpu.sync_copy(deg, out_hbm.at[wid, 0])


def _dis_body(p_ref, dis_ref):
    deg = jnp.sum(p_ref[...], axis=0, keepdims=True)
    dis_ref[...] = jnp.where(
        deg > 0.0, lax.rsqrt(jnp.maximum(deg, 1e-12)), 0.0)


_dis_tc = pl.pallas_call(
    _dis_body,
    out_shape=jax.ShapeDtypeStruct((1, _N), jnp.float32),
)


def _edge_norm_body(dis_hbm, src_hbm, dst_hbm, w_hbm, out_hbm,
                    disv, srcv, dstv, wv, nv):
    c = lax.axis_index("c")
    s = lax.axis_index("s")
    wid = s * _NC + c
    pltpu.sync_copy(dis_hbm, disv)
    pltpu.sync_copy(src_hbm.at[wid, 0], srcv)
    pltpu.sync_copy(dst_hbm.at[wid, 0], dstv)
    pltpu.sync_copy(w_hbm.at[wid, 0], wv)

    def _grp(i, _):
        sl = pl.ds(i * _L, _L)
        a = plsc.load_gather(disv, [srcv[sl]])
        b = plsc.load_gather(disv, [dstv[sl]])
        nv[sl] = a * wv[sl] * b
        return 0

    lax.fori_loop(0, _GP, _grp, 0)
    pltpu.sync_copy(nv, out_hbm.at[wid, 0])


_NB = 4
_PF = 2
_NG = 2 * _GP
_NGM = (_NG // _NB) * _NB


def _agg_body(t0_hbm, t1_hbm, src_hbm, dst_hbm, nrm_hbm, agg0_hbm, agg1_hbm,
              srcv, dstv, nrmv, rows, acc, gsem, ssem):
    c = lax.axis_index("c")
    s = lax.axis_index("s")

    z16 = jnp.zeros((_L,), jnp.float32)
    for i in range(_L):
        for k in range(_HH // _L):
            rows[0, i, pl.ds(k * _L, _L)] = z16
    r0 = s * _RQ
    for q in range(_RQL // _L):
        pltpu.sync_copy(rows.at[0], acc.at[pl.ds(r0 + q * _L, _L)])
    plsc.subcore_barrier()

    pltpu.sync_copy(src_hbm.at[s * 2, 0], srcv.at[pl.ds(0, _EPW)])
    pltpu.sync_copy(src_hbm.at[s * 2 + 1, 0], srcv.at[pl.ds(_EPW, _EPW)])
    pltpu.sync_copy(dst_hbm.at[s * 2, 0], dstv.at[pl.ds(0, _EPW)])
    pltpu.sync_copy(dst_hbm.at[s * 2 + 1, 0], dstv.at[pl.ds(_EPW, _EPW)])
    pltpu.sync_copy(nrm_hbm.at[s * 2, 0], nrmv.at[pl.ds(0, _EPW)])
    pltpu.sync_copy(nrm_hbm.at[s * 2 + 1, 0], nrmv.at[pl.ds(_EPW, _EPW)])

    t_hbm = t0_hbm

    def _gather(g, j):
        s16 = srcv[pl.ds(g * _L, _L)]

        @pl.when(c == 0)
        def _():
            pltpu.async_copy(t0_hbm.at[s16], rows.at[j], gsem)

        @pl.when(c == 1)
        def _():
            pltpu.async_copy(t1_hbm.at[s16], rows.at[j], gsem)

    def _wait_gather(j):
        z = jnp.zeros((_L,), jnp.int32)
        pltpu.make_async_copy(t_hbm.at[z], rows.at[j], gsem).wait()

    def _scale_scatter(g, j):
        base = g * _L
        for i2 in range(_L):
            eidx = (jnp.zeros((_L,), jnp.int32) + base + i2)
            spl = plsc.load_gather(nrmv, [eidx])
            for k in range(_HH // _L):
                ksl = pl.ds(k * _L, _L)
                rows[j, i2, ksl] = rows[j, i2, ksl] * spl
        d16 = dstv[pl.ds(g * _L, _L)]
        pltpu.async_copy(rows.at[j], acc.at[d16], ssem, add=True)

    def _drain_scatter(j):
        z = jnp.zeros((_L,), jnp.int32)
        pltpu.make_async_copy(rows.at[j], acc.at[z], ssem).wait()

    for j in range(_PF):
        _gather(j, j)

    def _chunk(m, _):
        g0 = m * _NB
        for jj in range(_NB):
            g = g0 + jj

            @pl.when(g >= _PF)
            def _():
                _drain_scatter((jj + _PF) % _NB)

            _gather(g + _PF, (jj + _PF) % _NB)
            _wait_gather(jj)
            _scale_scatter(g, jj)
        return 0

    lax.fori_loop(0, _NGM // _NB, _chunk, 0)

    for g in range(_NGM, _NG):
        jj = g % _NB
        _wait_gather(jj)
        _scale_scatter(g, jj)
    for jj in range(_PF + (_NG - _NGM)):
        _drain_scatter(jj)

    plsc.subcore_barrier()

    @pl.when(c == 0)
    def _():
        pltpu.sync_copy(acc.at[pl.ds(s * _RQ, _RQ)],
                        agg0_hbm.at[pl.ds(s * _RQ, _RQ)])

    @pl.when(c == 1)
    def _():
        pltpu.sync_copy(acc.at[pl.ds(s * _RQ, _RQ)],
                        agg1_hbm.at[pl.ds(s * _RQ, _RQ)])

    tail = _RQ * (_NS - 1)

    @pl.when(jnp.logical_and(c == 0, s == _NS - 1))
    def _():
        pltpu.sync_copy(acc.at[pl.ds(tail + _RQ, _RQL - _RQ)],
                        agg0_hbm.at[pl.ds(tail + _RQ, _RQL - _RQ)])

    @pl.when(jnp.logical_and(c == 1, s == _NS - 1))
    def _():
        pltpu.sync_copy(acc.at[pl.ds(tail + _RQ, _RQL - _RQ)],
                        agg1_hbm.at[pl.ds(tail + _RQ, _RQL - _RQ)])


@functools.cache
def _sc_kernels():
    mesh = plsc.VectorSubcoreMesh(core_axis_name="c", subcore_axis_name="s",
                                  num_cores=_NC, num_subcores=_NS)
    params = pltpu.CompilerParams(needs_layout_passes=False)
    deg = pl.kernel(
        _deg_partials_body,
        out_type=jax.ShapeDtypeStruct((_NW, 1, _N), jnp.float32),
        mesh=mesh,
        compiler_params=params,
        scratch_types=[
            pltpu.VMEM((_EPW,), jnp.int32),
            pltpu.VMEM((_EPW,), jnp.float32),
            pltpu.VMEM((_N,), jnp.float32),
        ],
    )
    enorm = pl.kernel(
        _edge_norm_body,
        out_type=jax.ShapeDtypeStruct((_NW, 1, _EPW), jnp.float32),
        mesh=mesh,
        compiler_params=params,
        scratch_types=[
            pltpu.VMEM((_N,), jnp.float32),
            pltpu.VMEM((_EPW,), jnp.int32),
            pltpu.VMEM((_EPW,), jnp.int32),
            pltpu.VMEM((_EPW,), jnp.float32),
            pltpu.VMEM((_EPW,), jnp.float32),
        ],
    )
    agg = pl.kernel(
        _agg_body,
        out_type=[jax.ShapeDtypeStruct((_N, _HH), jnp.float32)] * 2,
        mesh=mesh,
        compiler_params=params,
        scratch_types=[
            pltpu.VMEM((2 * _EPW,), jnp.int32),
            pltpu.VMEM((2 * _EPW,), jnp.int32),
            pltpu.VMEM((2 * _EPW,), jnp.float32),
            pltpu.VMEM((_NB, _L, _HH), jnp.float32),
            pltpu.VMEM_SHARED((_N, _HH), jnp.float32),
            pltpu.SemaphoreType.DMA,
            pltpu.SemaphoreType.DMA,
        ],
    )
    return deg, enorm, agg


_RB = 2000


def _mm0_body(x_ref, w_ref, t0_ref, t1_ref):
    t = lax.dot_general(x_ref[...], w_ref[...], (((1,), (1,)), ((), ())),
                        preferred_element_type=jnp.float32)
    t0_ref[...] = t[:, :_HH]
    t1_ref[...] = t[:, _HH:]


_mm0 = pl.pallas_call(
    _mm0_body,
    grid=(_N // _RB,),
    in_specs=[
        pl.BlockSpec((_RB, _D), lambda i: (i, 0)),
        pl.BlockSpec((_H, _D), lambda i: (0, 0)),
    ],
    out_specs=[pl.BlockSpec((_RB, _HH), lambda i: (i, 0))] * 2,
    out_shape=[jax.ShapeDtypeStruct((_N, _HH), jnp.float32)] * 2,
)


def _cm_body(a0_ref, a1_ref, b_ref, w_ref, t0_ref, t1_ref):
    b = b_ref[...]
    h0 = jnp.maximum(a0_ref[...] + b[:, :_HH], 0.0)
    h1 = jnp.maximum(a1_ref[...] + b[:, _HH:], 0.0)
    w = w_ref[...]
    t = (lax.dot_general(h0, w[:, :_HH], (((1,), (1,)), ((), ())),
                         preferred_element_type=jnp.float32)
         + lax.dot_general(h1, w[:, _HH:], (((1,), (1,)), ((), ())),
                           preferred_element_type=jnp.float32))
    t0_ref[...] = t[:, :_HH]
    t1_ref[...] = t[:, _HH:]


_cmm = pl.pallas_call(
    _cm_body,
    grid=(_N // _RB,),
    in_specs=[
        pl.BlockSpec((_RB, _HH), lambda i: (i, 0)),
        pl.BlockSpec((_RB, _HH), lambda i: (i, 0)),
        pl.BlockSpec((1, _H), lambda i: (0, 0)),
        pl.BlockSpec((_H, _H), lambda i: (0, 0)),
    ],
    out_specs=[pl.BlockSpec((_RB, _HH), lambda i: (i, 0))] * 2,
    out_shape=[jax.ShapeDtypeStruct((_N, _HH), jnp.float32)] * 2,
)


def _fin_body(a0_ref, a1_ref, b_ref, o_ref):
    b = b_ref[...]
    o_ref[:, :_HH] = a0_ref[...] + b[:, :_HH]
    o_ref[:, _HH:] = a1_ref[...] + b[:, _HH:]


_fin = pl.pallas_call(
    _fin_body,
    grid=(_N // _RB,),
    in_specs=[
        pl.BlockSpec((_RB, _HH), lambda i: (i, 0)),
        pl.BlockSpec((_RB, _HH), lambda i: (i, 0)),
        pl.BlockSpec((1, _H), lambda i: (0, 0)),
    ],
    out_specs=pl.BlockSpec((_RB, _H), lambda i: (i, 0)),
    out_shape=jax.ShapeDtypeStruct((_N, _H), jnp.float32),
)


def kernel(x, edge_index, edge_weight, batch,
           W0, b0, W1, b1, W2, b2, W3, b3, W4, b4):
    del batch
    loop = jnp.arange(_N, dtype=jnp.int32)
    padi = jnp.zeros((_EP - _ET,), jnp.int32)
    src_p = jnp.concatenate([edge_index[0], loop, padi]).reshape(_NW, 1, _EPW)
    dst_p = jnp.concatenate([edge_index[1], loop, padi]).reshape(_NW, 1, _EPW)
    w_p = jnp.concatenate([
        edge_weight.astype(jnp.float32),
        jnp.ones((_N,), jnp.float32),
        jnp.zeros((_EP - _ET,), jnp.float32),
    ]).reshape(_NW, 1, _EPW)

    _deg_partials, _edge_norm, _agg = _sc_kernels()
    partials = _deg_partials(dst_p, w_p)
    dis = _dis_tc(partials.reshape(_NW, _N)).reshape(_N)
    nrm = _edge_norm(dis, src_p, dst_p, w_p)

    t0, t1 = _mm0(x, W0)
    a0, a1 = _agg(t0, t1, src_p, dst_p, nrm)
    for (bb, WW) in ((b0, W1), (b1, W2), (b2, W3), (b3, W4)):
        t0, t1 = _cmm(a0, a1, bb.reshape(1, _H), WW)
        a0, a1 = _agg(t0, t1, src_p, dst_p, nrm)
    return _fin(a0, a1, b4.reshape(1, _H))


# speedup vs baseline: 1.2046x; 1.0005x over previous
import functools

import jax
import jax.numpy as jnp
from jax import lax
from jax.experimental import pallas as pl
from jax.experimental.pallas import tpu as pltpu
from jax.experimental.pallas import tpu_sc as plsc

_N = 10000
_E = 160000
_D = 256
_H = 256
_HH = _H // 2

_NC = 2
_NS = 16
_L = 16
_NW = _NC * _NS

_ET = _E + _N
_GP = -(-_ET // (_NW * _L))
_EPW = _GP * _L
_EP = _NW * _EPW

_GE = 32
_EPT = 2 * _EPW
_GT = _EPT // _GE

_RQ = 624
_RQL = _N - _RQ * (_NS - 1)


def _deg_partials_body(dst_hbm, w_hbm, out_hbm, dstv, wv, deg):
    c = lax.axis_index("c")
    s = lax.axis_index("s")
    wid = s * _NC + c
    pltpu.sync_copy(dst_hbm.at[wid, 0], dstv)
    pltpu.sync_copy(w_hbm.at[wid, 0], wv)
    zero = jnp.zeros((_L,), jnp.float32)

    def _zero(i, _):
        deg[pl.ds(i * _L, _L)] = zero
        return 0

    lax.fori_loop(0, _N // _L, _zero, 0)

    def _acc(i, _):
        d16 = dstv[pl.ds(i * _L, _L)]
        w16 = wv[pl.ds(i * _L, _L)]
        plsc.addupdate_scatter(deg, [d16], w16)
        return 0

    lax.fori_loop(0, _GP, _acc, 0)
    pltpu.sync_copy(deg, out_hbm.at[wid, 0])


def _dis_body(p_ref, dis_ref):
    deg = jnp.sum(p_ref[...], axis=0, keepdims=True)
    dis_ref[...] = jnp.where(
        deg > 0.0, lax.rsqrt(jnp.maximum(deg, 1e-12)), 0.0)


_dis_tc = pl.pallas_call(
    _dis_body,
    out_shape=jax.ShapeDtypeStruct((1, _N), jnp.float32),
)


def _edge_norm_body(dis_hbm, src_hbm, dst_hbm, w_hbm, out_hbm,
                    disv, srcv, dstv, wv, nv):
    c = lax.axis_index("c")
    s = lax.axis_index("s")
    wid = s * _NC + c
    pltpu.sync_copy(dis_hbm, disv)
    pltpu.sync_copy(src_hbm.at[wid, 0], srcv)
    pltpu.sync_copy(dst_hbm.at[wid, 0], dstv)
    pltpu.sync_copy(w_hbm.at[wid, 0], wv)

    def _grp(i, _):
        sl = pl.ds(i * _L, _L)
        a = plsc.load_gather(disv, [srcv[sl]])
        b = plsc.load_gather(disv, [dstv[sl]])
        nv[sl] = a * wv[sl] * b
        return 0

    lax.fori_loop(0, _GP, _grp, 0)
    pltpu.sync_copy(nv, out_hbm.at[wid, 0])


_NB = 4
_PF = 2
_NG = 2 * _GP
_NGM = (_NG // _NB) * _NB


def _agg_body(t0_hbm, t1_hbm, src_hbm, dst_hbm, nrm_hbm, agg0_hbm, agg1_hbm,
              srcv, dstv, nrmv, rows, acc, gsem, ssem):
    c = lax.axis_index("c")
    s = lax.axis_index("s")

    z16 = jnp.zeros((_L,), jnp.float32)
    for i in range(_L):
        for k in range(_HH // _L):
            rows[0, i, pl.ds(k * _L, _L)] = z16
    r0 = s * _RQ
    for q in range(_RQL // _L):
        pltpu.sync_copy(rows.at[0], acc.at[pl.ds(r0 + q * _L, _L)])
    plsc.subcore_barrier()

    pltpu.sync_copy(src_hbm.at[s * 2, 0], srcv.at[pl.ds(0, _EPW)])
    pltpu.sync_copy(src_hbm.at[s * 2 + 1, 0], srcv.at[pl.ds(_EPW, _EPW)])
    pltpu.sync_copy(dst_hbm.at[s * 2, 0], dstv.at[pl.ds(0, _EPW)])
    pltpu.sync_copy(dst_hbm.at[s * 2 + 1, 0], dstv.at[pl.ds(_EPW, _EPW)])
    pltpu.sync_copy(nrm_hbm.at[s * 2, 0], nrmv.at[pl.ds(0, _EPW)])
    pltpu.sync_copy(nrm_hbm.at[s * 2 + 1, 0], nrmv.at[pl.ds(_EPW, _EPW)])

    t_hbm = t0_hbm

    def _gather(g, j):
        s16 = srcv[pl.ds(g * _L, _L)]

        @pl.when(c == 0)
        def _():
            pltpu.async_copy(t0_hbm.at[s16], rows.at[j], gsem)

        @pl.when(c == 1)
        def _():
            pltpu.async_copy(t1_hbm.at[s16], rows.at[j], gsem)

    def _wait_gather(j):
        z = jnp.zeros((_L,), jnp.int32)
        pltpu.make_async_copy(t_hbm.at[z], rows.at[j], gsem).wait()

    def _scale_scatter(g, j):
        base = g * _L
        for i2 in range(_L):
            eidx = (jnp.zeros((_L,), jnp.int32) + base + i2)
            spl = plsc.load_gather(nrmv, [eidx])
            for k in range(_HH // _L):
                ksl = pl.ds(k * _L, _L)
                rows[j, i2, ksl] = rows[j, i2, ksl] * spl
        d16 = dstv[pl.ds(g * _L, _L)]
        pltpu.async_copy(rows.at[j], acc.at[d16], ssem, add=True)

    def _drain_scatter(j):
        z = jnp.zeros((_L,), jnp.int32)
        pltpu.make_async_copy(rows.at[j], acc.at[z], ssem).wait()

    for j in range(_PF):
        _gather(j, j)

    def _chunk(m, _):
        g0 = m * _NB
        for jj in range(_NB):
            g = g0 + jj

            @pl.when(g >= _PF)
            def _():
                _drain_scatter((jj + _PF) % _NB)

            _gather(g + _PF, (jj + _PF) % _NB)
            _wait_gather(jj)
            _scale_scatter(g, jj)
        return 0

    lax.fori_loop(0, _NGM // _NB, _chunk, 0)

    for g in range(_NGM, _NG):
        jj = g % _NB
        _wait_gather(jj)
        _scale_scatter(g, jj)
    for jj in range(_PF + (_NG - _NGM)):
        _drain_scatter(jj)

    plsc.subcore_barrier()

    @pl.when(c == 0)
    def _():
        pltpu.sync_copy(acc.at[pl.ds(s * _RQ, _RQ)],
                        agg0_hbm.at[pl.ds(s * _RQ, _RQ)])

    @pl.when(c == 1)
    def _():
        pltpu.sync_copy(acc.at[pl.ds(s * _RQ, _RQ)],
                        agg1_hbm.at[pl.ds(s * _RQ, _RQ)])

    tail = _RQ * (_NS - 1)

    @pl.when(jnp.logical_and(c == 0, s == _NS - 1))
    def _():
        pltpu.sync_copy(acc.at[pl.ds(tail + _RQ, _RQL - _RQ)],
                        agg0_hbm.at[pl.ds(tail + _RQ, _RQL - _RQ)])

    @pl.when(jnp.logical_and(c == 1, s == _NS - 1))
    def _():
        pltpu.sync_copy(acc.at[pl.ds(tail + _RQ, _RQL - _RQ)],
                        agg1_hbm.at[pl.ds(tail + _RQ, _RQL - _RQ)])


@functools.cache
def _sc_kernels():
    mesh = plsc.VectorSubcoreMesh(core_axis_name="c", subcore_axis_name="s",
                                  num_cores=_NC, num_subcores=_NS)
    params = pltpu.CompilerParams(needs_layout_passes=False)
    deg = pl.kernel(
        _deg_partials_body,
        out_type=jax.ShapeDtypeStruct((_NW, 1, _N), jnp.float32),
        mesh=mesh,
        compiler_params=params,
        scratch_types=[
            pltpu.VMEM((_EPW,), jnp.int32),
            pltpu.VMEM((_EPW,), jnp.float32),
            pltpu.VMEM((_N,), jnp.float32),
        ],
    )
    enorm = pl.kernel(
        _edge_norm_body,
        out_type=jax.ShapeDtypeStruct((_NW, 1, _EPW), jnp.float32),
        mesh=mesh,
        compiler_params=params,
        scratch_types=[
            pltpu.VMEM((_N,), jnp.float32),
            pltpu.VMEM((_EPW,), jnp.int32),
            pltpu.VMEM((_EPW,), jnp.int32),
            pltpu.VMEM((_EPW,), jnp.float32),
            pltpu.VMEM((_EPW,), jnp.float32),
        ],
    )
    agg = pl.kernel(
        _agg_body,
        out_type=[jax.ShapeDtypeStruct((_N, _HH), jnp.float32)] * 2,
        mesh=mesh,
        compiler_params=params,
        scratch_types=[
            pltpu.VMEM((2 * _EPW,), jnp.int32),
            pltpu.VMEM((2 * _EPW,), jnp.int32),
            pltpu.VMEM((2 * _EPW,), jnp.float32),
            pltpu.VMEM((_NB, _L, _HH), jnp.float32),
            pltpu.VMEM_SHARED((_N, _HH), jnp.float32),
            pltpu.SemaphoreType.DMA,
            pltpu.SemaphoreType.DMA,
        ],
    )
    return deg, enorm, agg


_RB = 2000


def _mm0_body(x_ref, w_ref, t0_ref, t1_ref):
    t = lax.dot_general(x_ref[...], w_ref[...], (((1,), (1,)), ((), ())),
                        preferred_element_type=jnp.float32)
    t0_ref[...] = t[:, :_HH]
    t1_ref[...] = t[:, _HH:]


_mm0 = pl.pallas_call(
    _mm0_body,
    grid=(_N // _RB,),
    in_specs=[
        pl.BlockSpec((_RB, _D), lambda i: (i, 0)),
        pl.BlockSpec((_H, _D), lambda i: (0, 0)),
    ],
    out_specs=[pl.BlockSpec((_RB, _HH), lambda i: (i, 0))] * 2,
    out_shape=[jax.ShapeDtypeStruct((_N, _HH), jnp.float32)] * 2,
)


def _cm_body(a0_ref, a1_ref, b_ref, w_ref, t0_ref, t1_ref):
    b = b_ref[...]
    h0 = jnp.maximum(a0_ref[...] + b[:, :_HH], 0.0)
    h1 = jnp.maximum(a1_ref[...] + b[:, _HH:], 0.0)
    w = w_ref[...]
    t = (lax.dot_general(h0, w[:, :_HH], (((1,), (1,)), ((), ())),
                         preferred_element_type=jnp.float32)
         + lax.dot_general(h1, w[:, _HH:], (((1,), (1,)), ((), ())),
                           preferred_element_type=jnp.float32))
    t0_ref[...] = t[:, :_HH]
    t1_ref[...] = t[:, _HH:]


_cmm = pl.pallas_call(
    _cm_body,
    grid=(_N // _RB,),
    in_specs=[
        pl.BlockSpec((_RB, _HH), lambda i: (i, 0)),
        pl.BlockSpec((_RB, _HH), lambda i: (i, 0)),
        pl.BlockSpec((1, _H), lambda i: (0, 0)),
        pl.BlockSpec((_H, _H), lambda i: (0, 0)),
    ],
    out_specs=[pl.BlockSpec((_RB, _HH), lambda i: (i, 0))] * 2,
    out_shape=[jax.ShapeDtypeStruct((_N, _HH), jnp.float32)] * 2,
)


def _fin_body(a0_ref, a1_ref, b_ref, o_ref):
    b = b_ref[...]
    o_ref[:, :_HH] = a0_ref[...] + b[:, :_HH]
    o_ref[:, _HH:] = a1_ref[...] + b[:, _HH:]


_fin = pl.pallas_call(
    _fin_body,
    grid=(_N // _RB,),
    in_specs=[
        pl.BlockSpec((_RB, _HH), lambda i: (i, 0)),
        pl.BlockSpec((_RB, _HH), lambda i: (i, 0)),
        pl.BlockSpec((1, _H), lambda i: (0, 0)),
    ],
    out_specs=pl.BlockSpec((_RB, _H), lambda i: (i, 0)),
    out_shape=jax.ShapeDtypeStruct((_N, _H), jnp.float32),
)


def kernel(x, edge_index, edge_weight, batch,
           W0, b0, W1, b1, W2, b2, W3, b3, W4, b4):
    del batch
    loop = jnp.arange(_N, dtype=jnp.int32)
    padi = jnp.zeros((_EP - _ET,), jnp.int32)
    src_f = jnp.concatenate([edge_index[0], loop, padi])
    dst_f = jnp.concatenate([edge_index[1], loop, padi])
    w_p = jnp.concatenate([
        edge_weight.astype(jnp.float32),
        jnp.ones((_N,), jnp.float32),
        jnp.zeros((_EP - _ET,), jnp.float32),
    ]).reshape(_NW, 1, _EPW)
    src_p = src_f.reshape(_NW, 1, _EPW)
    dst_p = dst_f.reshape(_NW, 1, _EPW)

    _deg_partials, _edge_norm, _agg = _sc_kernels()
    partials = _deg_partials(dst_p, w_p)
    dis = _dis_tc(partials.reshape(_NW, _N)).reshape(_N)
    nrm = _edge_norm(dis, src_p, dst_p, w_p)

    t0, t1 = _mm0(x, W0)
    a0, a1 = _agg(t0, t1, src_p, dst_p, nrm)
    for (bb, WW) in ((b0, W1), (b1, W2), (b2, W3), (b3, W4)):
        t0, t1 = _cmm(a0, a1, bb.reshape(1, _H), WW)
        a0, a1 = _agg(t0, t1, src_p, dst_p, nrm)
    return _fin(a0, a1, b4.reshape(1, _H))


# speedup vs baseline: 1.2600x; 1.0460x over previous
import functools

import jax
import jax.numpy as jnp
from jax import lax
from jax.experimental import pallas as pl
from jax.experimental.pallas import tpu as pltpu
from jax.experimental.pallas import tpu_sc as plsc

_N = 10000
_E = 160000
_D = 256
_H = 256
_HH = _H // 2

_NC = 2
_NS = 16
_L = 16
_NW = _NC * _NS

_ET = _E + _N
_GP = -(-_ET // (_NW * _L))
_EPW = _GP * _L
_EP = _NW * _EPW

_GE = 32
_EPT = 2 * _EPW
_GT = _EPT // _GE

_RQ = 624
_RQL = _N - _RQ * (_NS - 1)


def _deg_partials_body(dst_hbm, w_hbm, out_hbm, dstv, wv, deg):
    c = lax.axis_index("c")
    s = lax.axis_index("s")
    wid = s * _NC + c
    pltpu.sync_copy(dst_hbm.at[wid, 0], dstv)
    pltpu.sync_copy(w_hbm.at[wid, 0], wv)
    zero = jnp.zeros((_L,), jnp.float32)

    def _zero(i, _):
        deg[pl.ds(i * _L, _L)] = zero
        return 0

    lax.fori_loop(0, _N // _L, _zero, 0)

    def _acc(i, _):
        d16 = dstv[pl.ds(i * _L, _L)]
        w16 = wv[pl.ds(i * _L, _L)]
        plsc.addupdate_scatter(deg, [d16], w16)
        return 0

    lax.fori_loop(0, _GP, _acc, 0)
    pltpu.sync_copy(deg, out_hbm.at[wid, 0])


def _dis_body(p_ref, dis_ref):
    deg = jnp.sum(p_ref[...], axis=0, keepdims=True)
    dis_ref[...] = jnp.where(
        deg > 0.0, lax.rsqrt(jnp.maximum(deg, 1e-12)), 0.0)


_dis_tc = pl.pallas_call(
    _dis_body,
    out_shape=jax.ShapeDtypeStruct((1, _N), jnp.float32),
)


def _edge_norm_body(dis_hbm, src_hbm, dst_hbm, w_hbm, out_hbm,
                    disv, srcv, dstv, wv, nv):
    c = lax.axis_index("c")
    s = lax.axis_index("s")
    wid = s * _NC + c
    pltpu.sync_copy(dis_hbm, disv)
    pltpu.sync_copy(src_hbm.at[wid, 0], srcv)
    pltpu.sync_copy(dst_hbm.at[wid, 0], dstv)
    pltpu.sync_copy(w_hbm.at[wid, 0], wv)

    def _grp(i, _):
        sl = pl.ds(i * _L, _L)
        a = plsc.load_gather(disv, [srcv[sl]])
        b = plsc.load_gather(disv, [dstv[sl]])
        nv[sl] = a * wv[sl] * b
        return 0

    lax.fori_loop(0, _GP, _grp, 0)
    pltpu.sync_copy(nv, out_hbm.at[wid, 0])


_NB = 4
_PF = 2
_NG = 2 * _GP
_NGM = (_NG // _NB) * _NB


def _agg_body(t0_hbm, t1_hbm, src_hbm, dst_hbm, nrm_hbm, agg0_hbm, agg1_hbm,
              srcv, dstv, nrmv, rows, acc, gsem, ssem):
    c = lax.axis_index("c")
    s = lax.axis_index("s")

    z16 = jnp.zeros((_L,), jnp.float32)
    for i in range(_L):
        for k in range(_HH // _L):
            rows[0, i, pl.ds(k * _L, _L)] = z16
    r0 = s * _RQ
    for q in range(_RQL // _L):
        pltpu.sync_copy(rows.at[0], acc.at[pl.ds(r0 + q * _L, _L)])
    plsc.subcore_barrier()

    pltpu.sync_copy(src_hbm.at[s * 2, 0], srcv.at[pl.ds(0, _EPW)])
    pltpu.sync_copy(src_hbm.at[s * 2 + 1, 0], srcv.at[pl.ds(_EPW, _EPW)])
    pltpu.sync_copy(dst_hbm.at[s * 2, 0], dstv.at[pl.ds(0, _EPW)])
    pltpu.sync_copy(dst_hbm.at[s * 2 + 1, 0], dstv.at[pl.ds(_EPW, _EPW)])
    pltpu.sync_copy(nrm_hbm.at[s * 2, 0], nrmv.at[pl.ds(0, _EPW)])
    pltpu.sync_copy(nrm_hbm.at[s * 2 + 1, 0], nrmv.at[pl.ds(_EPW, _EPW)])

    t_hbm = t0_hbm

    def _gather(g, j):
        s16 = srcv[pl.ds(g * _L, _L)]

        @pl.when(c == 0)
        def _():
            pltpu.async_copy(t0_hbm.at[s16], rows.at[j], gsem)

        @pl.when(c == 1)
        def _():
            pltpu.async_copy(t1_hbm.at[s16], rows.at[j], gsem)

    def _wait_gather(j):
        z = jnp.zeros((_L,), jnp.int32)
        pltpu.make_async_copy(t_hbm.at[z], rows.at[j], gsem).wait()

    def _scale_scatter(g, j):
        n16 = nrmv[pl.ds(g * _L, _L)]
        dnums = lax.GatherDimensionNumbers(
            offset_dims=(), collapsed_slice_dims=(0,), start_index_map=(0,))
        for i2 in range(_L):
            idx = jnp.full((_L, 1), i2, jnp.int32)
            spl = lax.gather(n16, idx, dnums, (1,),
                             mode=lax.GatherScatterMode.PROMISE_IN_BOUNDS)
            for k in range(_HH // _L):
                ksl = pl.ds(k * _L, _L)
                rows[j, i2, ksl] = rows[j, i2, ksl] * spl
        d16 = dstv[pl.ds(g * _L, _L)]
        pltpu.async_copy(rows.at[j], acc.at[d16], ssem, add=True)

    def _drain_scatter(j):
        z = jnp.zeros((_L,), jnp.int32)
        pltpu.make_async_copy(rows.at[j], acc.at[z], ssem).wait()

    for j in range(_PF):
        _gather(j, j)

    def _chunk(m, _):
        g0 = m * _NB
        for jj in range(_NB):
            g = g0 + jj

            @pl.when(g >= _PF)
            def _():
                _drain_scatter((jj + _PF) % _NB)

            _gather(g + _PF, (jj + _PF) % _NB)
            _wait_gather(jj)
            _scale_scatter(g, jj)
        return 0

    lax.fori_loop(0, _NGM // _NB, _chunk, 0)

    for g in range(_NGM, _NG):
        jj = g % _NB
        _wait_gather(jj)
        _scale_scatter(g, jj)
    for jj in range(_PF + (_NG - _NGM)):
        _drain_scatter(jj)

    plsc.subcore_barrier()

    @pl.when(c == 0)
    def _():
        pltpu.sync_copy(acc.at[pl.ds(s * _RQ, _RQ)],
                        agg0_hbm.at[pl.ds(s * _RQ, _RQ)])

    @pl.when(c == 1)
    def _():
        pltpu.sync_copy(acc.at[pl.ds(s * _RQ, _RQ)],
                        agg1_hbm.at[pl.ds(s * _RQ, _RQ)])

    tail = _RQ * (_NS - 1)

    @pl.when(jnp.logical_and(c == 0, s == _NS - 1))
    def _():
        pltpu.sync_copy(acc.at[pl.ds(tail + _RQ, _RQL - _RQ)],
                        agg0_hbm.at[pl.ds(tail + _RQ, _RQL - _RQ)])

    @pl.when(jnp.logical_and(c == 1, s == _NS - 1))
    def _():
        pltpu.sync_copy(acc.at[pl.ds(tail + _RQ, _RQL - _RQ)],
                        agg1_hbm.at[pl.ds(tail + _RQ, _RQL - _RQ)])


@functools.cache
def _sc_kernels():
    mesh = plsc.VectorSubcoreMesh(core_axis_name="c", subcore_axis_name="s",
                                  num_cores=_NC, num_subcores=_NS)
    params = pltpu.CompilerParams(needs_layout_passes=False)
    deg = pl.kernel(
        _deg_partials_body,
        out_type=jax.ShapeDtypeStruct((_NW, 1, _N), jnp.float32),
        mesh=mesh,
        compiler_params=params,
        scratch_types=[
            pltpu.VMEM((_EPW,), jnp.int32),
            pltpu.VMEM((_EPW,), jnp.float32),
            pltpu.VMEM((_N,), jnp.float32),
        ],
    )
    enorm = pl.kernel(
        _edge_norm_body,
        out_type=jax.ShapeDtypeStruct((_NW, 1, _EPW), jnp.float32),
        mesh=mesh,
        compiler_params=params,
        scratch_types=[
            pltpu.VMEM((_N,), jnp.float32),
            pltpu.VMEM((_EPW,), jnp.int32),
            pltpu.VMEM((_EPW,), jnp.int32),
            pltpu.VMEM((_EPW,), jnp.float32),
            pltpu.VMEM((_EPW,), jnp.float32),
        ],
    )
    agg = pl.kernel(
        _agg_body,
        out_type=[jax.ShapeDtypeStruct((_N, _HH), jnp.float32)] * 2,
        mesh=mesh,
        compiler_params=params,
        scratch_types=[
            pltpu.VMEM((2 * _EPW,), jnp.int32),
            pltpu.VMEM((2 * _EPW,), jnp.int32),
            pltpu.VMEM((2 * _EPW,), jnp.float32),
            pltpu.VMEM((_NB, _L, _HH), jnp.float32),
            pltpu.VMEM_SHARED((_N, _HH), jnp.float32),
            pltpu.SemaphoreType.DMA,
            pltpu.SemaphoreType.DMA,
        ],
    )
    return deg, enorm, agg


_RB = 2000


def _mm0_body(x_ref, w_ref, t0_ref, t1_ref):
    t = lax.dot_general(x_ref[...], w_ref[...], (((1,), (1,)), ((), ())),
                        preferred_element_type=jnp.float32)
    t0_ref[...] = t[:, :_HH]
    t1_ref[...] = t[:, _HH:]


_mm0 = pl.pallas_call(
    _mm0_body,
    grid=(_N // _RB,),
    in_specs=[
        pl.BlockSpec((_RB, _D), lambda i: (i, 0)),
        pl.BlockSpec((_H, _D), lambda i: (0, 0)),
    ],
    out_specs=[pl.BlockSpec((_RB, _HH), lambda i: (i, 0))] * 2,
    out_shape=[jax.ShapeDtypeStruct((_N, _HH), jnp.float32)] * 2,
)


def _cm_body(a0_ref, a1_ref, b_ref, w_ref, t0_ref, t1_ref):
    b = b_ref[...]
    h0 = jnp.maximum(a0_ref[...] + b[:, :_HH], 0.0)
    h1 = jnp.maximum(a1_ref[...] + b[:, _HH:], 0.0)
    w = w_ref[...]
    t = (lax.dot_general(h0, w[:, :_HH], (((1,), (1,)), ((), ())),
                         preferred_element_type=jnp.float32)
         + lax.dot_general(h1, w[:, _HH:], (((1,), (1,)), ((), ())),
                           preferred_element_type=jnp.float32))
    t0_ref[...] = t[:, :_HH]
    t1_ref[...] = t[:, _HH:]


_cmm = pl.pallas_call(
    _cm_body,
    grid=(_N // _RB,),
    in_specs=[
        pl.BlockSpec((_RB, _HH), lambda i: (i, 0)),
        pl.BlockSpec((_RB, _HH), lambda i: (i, 0)),
        pl.BlockSpec((1, _H), lambda i: (0, 0)),
        pl.BlockSpec((_H, _H), lambda i: (0, 0)),
    ],
    out_specs=[pl.BlockSpec((_RB, _HH), lambda i: (i, 0))] * 2,
    out_shape=[jax.ShapeDtypeStruct((_N, _HH), jnp.float32)] * 2,
)


def _fin_body(a0_ref, a1_ref, b_ref, o_ref):
    b = b_ref[...]
    o_ref[:, :_HH] = a0_ref[...] + b[:, :_HH]
    o_ref[:, _HH:] = a1_ref[...] + b[:, _HH:]


_fin = pl.pallas_call(
    _fin_body,
    grid=(_N // _RB,),
    in_specs=[
        pl.BlockSpec((_RB, _HH), lambda i: (i, 0)),
        pl.BlockSpec((_RB, _HH), lambda i: (i, 0)),
        pl.BlockSpec((1, _H), lambda i: (0, 0)),
    ],
    out_specs=pl.BlockSpec((_RB, _H), lambda i: (i, 0)),
    out_shape=jax.ShapeDtypeStruct((_N, _H), jnp.float32),
)


def kernel(x, edge_index, edge_weight, batch,
           W0, b0, W1, b1, W2, b2, W3, b3, W4, b4):
    del batch
    loop = jnp.arange(_N, dtype=jnp.int32)
    padi = jnp.zeros((_EP - _ET,), jnp.int32)
    src_f = jnp.concatenate([edge_index[0], loop, padi])
    dst_f = jnp.concatenate([edge_index[1], loop, padi])
    w_p = jnp.concatenate([
        edge_weight.astype(jnp.float32),
        jnp.ones((_N,), jnp.float32),
        jnp.zeros((_EP - _ET,), jnp.float32),
    ]).reshape(_NW, 1, _EPW)
    src_p = src_f.reshape(_NW, 1, _EPW)
    dst_p = dst_f.reshape(_NW, 1, _EPW)

    _deg_partials, _edge_norm, _agg = _sc_kernels()
    partials = _deg_partials(dst_p, w_p)
    dis = _dis_tc(partials.reshape(_NW, _N)).reshape(_N)
    nrm = _edge_norm(dis, src_p, dst_p, w_p)

    t0, t1 = _mm0(x, W0)
    a0, a1 = _agg(t0, t1, src_p, dst_p, nrm)
    for (bb, WW) in ((b0, W1), (b1, W2), (b2, W3), (b3, W4)):
        t0, t1 = _cmm(a0, a1, bb.reshape(1, _H), WW)
        a0, a1 = _agg(t0, t1, src_p, dst_p, nrm)
    return _fin(a0, a1, b4.reshape(1, _H))


# speedup vs baseline: 1.2610x; 1.0008x over previous
import functools

import jax
import jax.numpy as jnp
from jax import lax
from jax.experimental import pallas as pl
from jax.experimental.pallas import tpu as pltpu
from jax.experimental.pallas import tpu_sc as plsc

_N = 10000
_E = 160000
_D = 256
_H = 256
_HH = _H // 2

_NC = 2
_NS = 16
_L = 16
_NW = _NC * _NS

_ET = _E + _N
_GP = -(-_ET // (_NW * _L))
_EPW = _GP * _L
_EP = _NW * _EPW

_GE = 32
_EPT = 2 * _EPW
_GT = _EPT // _GE

_RQ = 624
_RQL = _N - _RQ * (_NS - 1)


def _deg_partials_body(dst_hbm, w_hbm, out_hbm, dstv, wv, deg):
    c = lax.axis_index("c")
    s = lax.axis_index("s")
    wid = s * _NC + c
    pltpu.sync_copy(dst_hbm.at[wid, 0], dstv)
    pltpu.sync_copy(w_hbm.at[wid, 0], wv)
    zero = jnp.zeros((_L,), jnp.float32)

    def _zero(i, _):
        deg[pl.ds(i * _L, _L)] = zero
        return 0

    lax.fori_loop(0, _N // _L, _zero, 0)

    def _acc(i, _):
        d16 = dstv[pl.ds(i * _L, _L)]
        w16 = wv[pl.ds(i * _L, _L)]
        plsc.addupdate_scatter(deg, [d16], w16)
        return 0

    lax.fori_loop(0, _GP, _acc, 0)
    pltpu.sync_copy(deg, out_hbm.at[wid, 0])


def _dis_body(p_ref, dis_ref):
    deg = jnp.sum(p_ref[...], axis=0, keepdims=True)
    dis_ref[...] = jnp.where(
        deg > 0.0, lax.rsqrt(jnp.maximum(deg, 1e-12)), 0.0)


_dis_tc = pl.pallas_call(
    _dis_body,
    out_shape=jax.ShapeDtypeStruct((1, _N), jnp.float32),
)


def _edge_norm_body(dis_hbm, src_hbm, dst_hbm, w_hbm, out_hbm,
                    disv, srcv, dstv, wv, nv):
    c = lax.axis_index("c")
    s = lax.axis_index("s")
    wid = s * _NC + c
    pltpu.sync_copy(dis_hbm, disv)
    pltpu.sync_copy(src_hbm.at[wid, 0], srcv)
    pltpu.sync_copy(dst_hbm.at[wid, 0], dstv)
    pltpu.sync_copy(w_hbm.at[wid, 0], wv)

    def _grp(i, _):
        sl = pl.ds(i * _L, _L)
        a = plsc.load_gather(disv, [srcv[sl]])
        b = plsc.load_gather(disv, [dstv[sl]])
        nv[sl] = a * wv[sl] * b
        return 0

    lax.fori_loop(0, _GP, _grp, 0)
    pltpu.sync_copy(nv, out_hbm.at[wid, 0])


_NB = 4
_PF = 2
_NG = 2 * _GP
_NGM = (_NG // _NB) * _NB


def _agg_body(t0_hbm, t1_hbm, src_hbm, dst_hbm, nrm_hbm, agg0_hbm, agg1_hbm,
              srcv, dstv, nrmv, rows, acc, gsem, ssem):
    c = lax.axis_index("c")
    s = lax.axis_index("s")

    z16 = jnp.zeros((_L,), jnp.float32)
    for i in range(_L):
        for k in range(_HH // _L):
            rows[0, i, pl.ds(k * _L, _L)] = z16
    r0 = s * _RQ
    for q in range(_RQL // _L):
        pltpu.sync_copy(rows.at[0], acc.at[pl.ds(r0 + q * _L, _L)])
    plsc.subcore_barrier()

    pltpu.sync_copy(src_hbm.at[s * 2, 0], srcv.at[pl.ds(0, _EPW)])
    pltpu.sync_copy(src_hbm.at[s * 2 + 1, 0], srcv.at[pl.ds(_EPW, _EPW)])
    pltpu.sync_copy(dst_hbm.at[s * 2, 0], dstv.at[pl.ds(0, _EPW)])
    pltpu.sync_copy(dst_hbm.at[s * 2 + 1, 0], dstv.at[pl.ds(_EPW, _EPW)])
    pltpu.sync_copy(nrm_hbm.at[s * 2, 0], nrmv.at[pl.ds(0, _EPW)])
    pltpu.sync_copy(nrm_hbm.at[s * 2 + 1, 0], nrmv.at[pl.ds(_EPW, _EPW)])

    t_hbm = t0_hbm

    def _gather(g, j):
        s16 = srcv[pl.ds(g * _L, _L)]

        @pl.when(c == 0)
        def _():
            pltpu.async_copy(t0_hbm.at[s16], rows.at[j], gsem)

        @pl.when(c == 1)
        def _():
            pltpu.async_copy(t1_hbm.at[s16], rows.at[j], gsem)

    def _wait_gather(j):
        z = jnp.zeros((_L,), jnp.int32)
        pltpu.make_async_copy(t_hbm.at[z], rows.at[j], gsem).wait()

    def _scale_scatter(g, j):
        n16 = nrmv[pl.ds(g * _L, _L)]
        dnums = lax.GatherDimensionNumbers(
            offset_dims=(), collapsed_slice_dims=(0,), start_index_map=(0,))
        for i2 in range(_L):
            idx = jnp.full((_L, 1), i2, jnp.int32)
            spl = lax.gather(n16, idx, dnums, (1,),
                             mode=lax.GatherScatterMode.PROMISE_IN_BOUNDS)
            for k in range(_HH // _L):
                ksl = pl.ds(k * _L, _L)
                rows[j, i2, ksl] = rows[j, i2, ksl] * spl
        d16 = dstv[pl.ds(g * _L, _L)]
        pltpu.async_copy(rows.at[j], acc.at[d16], ssem, add=True)

    def _drain_scatter(j):
        z = jnp.zeros((_L,), jnp.int32)
        pltpu.make_async_copy(rows.at[j], acc.at[z], ssem).wait()

    for j in range(_PF):
        _gather(j, j)

    for jj in range(_NB):
        if jj >= _PF:
            _drain_scatter((jj + _PF) % _NB)
        _gather(jj + _PF, (jj + _PF) % _NB)
        _wait_gather(jj)
        _scale_scatter(jj, jj)

    def _chunk(m, _):
        g0 = m * _NB
        for jj in range(_NB):
            g = g0 + jj
            _drain_scatter((jj + _PF) % _NB)
            _gather(g + _PF, (jj + _PF) % _NB)
            _wait_gather(jj)
            _scale_scatter(g, jj)
        return 0

    lax.fori_loop(1, _NGM // _NB, _chunk, 0)

    for g in range(_NGM, _NG):
        jj = g % _NB
        _wait_gather(jj)
        _scale_scatter(g, jj)
    for jj in range(_PF + (_NG - _NGM)):
        _drain_scatter(jj)

    plsc.subcore_barrier()

    @pl.when(c == 0)
    def _():
        pltpu.sync_copy(acc.at[pl.ds(s * _RQ, _RQ)],
                        agg0_hbm.at[pl.ds(s * _RQ, _RQ)])

    @pl.when(c == 1)
    def _():
        pltpu.sync_copy(acc.at[pl.ds(s * _RQ, _RQ)],
                        agg1_hbm.at[pl.ds(s * _RQ, _RQ)])

    tail = _RQ * (_NS - 1)

    @pl.when(jnp.logical_and(c == 0, s == _NS - 1))
    def _():
        pltpu.sync_copy(acc.at[pl.ds(tail + _RQ, _RQL - _RQ)],
                        agg0_hbm.at[pl.ds(tail + _RQ, _RQL - _RQ)])

    @pl.when(jnp.logical_and(c == 1, s == _NS - 1))
    def _():
        pltpu.sync_copy(acc.at[pl.ds(tail + _RQ, _RQL - _RQ)],
                        agg1_hbm.at[pl.ds(tail + _RQ, _RQL - _RQ)])


@functools.cache
def _sc_kernels():
    mesh = plsc.VectorSubcoreMesh(core_axis_name="c", subcore_axis_name="s",
                                  num_cores=_NC, num_subcores=_NS)
    params = pltpu.CompilerParams(needs_layout_passes=False)
    deg = pl.kernel(
        _deg_partials_body,
        out_type=jax.ShapeDtypeStruct((_NW, 1, _N), jnp.float32),
        mesh=mesh,
        compiler_params=params,
        scratch_types=[
            pltpu.VMEM((_EPW,), jnp.int32),
            pltpu.VMEM((_EPW,), jnp.float32),
            pltpu.VMEM((_N,), jnp.float32),
        ],
    )
    enorm = pl.kernel(
        _edge_norm_body,
        out_type=jax.ShapeDtypeStruct((_NW, 1, _EPW), jnp.float32),
        mesh=mesh,
        compiler_params=params,
        scratch_types=[
            pltpu.VMEM((_N,), jnp.float32),
            pltpu.VMEM((_EPW,), jnp.int32),
            pltpu.VMEM((_EPW,), jnp.int32),
            pltpu.VMEM((_EPW,), jnp.float32),
            pltpu.VMEM((_EPW,), jnp.float32),
        ],
    )
    agg = pl.kernel(
        _agg_body,
        out_type=[jax.ShapeDtypeStruct((_N, _HH), jnp.float32)] * 2,
        mesh=mesh,
        compiler_params=params,
        scratch_types=[
            pltpu.VMEM((2 * _EPW,), jnp.int32),
            pltpu.VMEM((2 * _EPW,), jnp.int32),
            pltpu.VMEM((2 * _EPW,), jnp.float32),
            pltpu.VMEM((_NB, _L, _HH), jnp.float32),
            pltpu.VMEM_SHARED((_N, _HH), jnp.float32),
            pltpu.SemaphoreType.DMA,
            pltpu.SemaphoreType.DMA,
        ],
    )
    return deg, enorm, agg


_RB = 2000


def _mm0_body(x_ref, w_ref, t0_ref, t1_ref):
    t = lax.dot_general(x_ref[...], w_ref[...], (((1,), (1,)), ((), ())),
                        preferred_element_type=jnp.float32)
    t0_ref[...] = t[:, :_HH]
    t1_ref[...] = t[:, _HH:]


_mm0 = pl.pallas_call(
    _mm0_body,
    grid=(_N // _RB,),
    in_specs=[
        pl.BlockSpec((_RB, _D), lambda i: (i, 0)),
        pl.BlockSpec((_H, _D), lambda i: (0, 0)),
    ],
    out_specs=[pl.BlockSpec((_RB, _HH), lambda i: (i, 0))] * 2,
    out_shape=[jax.ShapeDtypeStruct((_N, _HH), jnp.float32)] * 2,
)


def _cm_body(a0_ref, a1_ref, b_ref, w_ref, t0_ref, t1_ref):
    b = b_ref[...]
    h0 = jnp.maximum(a0_ref[...] + b[:, :_HH], 0.0)
    h1 = jnp.maximum(a1_ref[...] + b[:, _HH:], 0.0)
    w = w_ref[...]
    t = (lax.dot_general(h0, w[:, :_HH], (((1,), (1,)), ((), ())),
                         preferred_element_type=jnp.float32)
         + lax.dot_general(h1, w[:, _HH:], (((1,), (1,)), ((), ())),
                           preferred_element_type=jnp.float32))
    t0_ref[...] = t[:, :_HH]
    t1_ref[...] = t[:, _HH:]


_cmm = pl.pallas_call(
    _cm_body,
    grid=(_N // _RB,),
    in_specs=[
        pl.BlockSpec((_RB, _HH), lambda i: (i, 0)),
        pl.BlockSpec((_RB, _HH), lambda i: (i, 0)),
        pl.BlockSpec((1, _H), lambda i: (0, 0)),
        pl.BlockSpec((_H, _H), lambda i: (0, 0)),
    ],
    out_specs=[pl.BlockSpec((_RB, _HH), lambda i: (i, 0))] * 2,
    out_shape=[jax.ShapeDtypeStruct((_N, _HH), jnp.float32)] * 2,
)


def _fin_body(a0_ref, a1_ref, b_ref, o_ref):
    b = b_ref[...]
    o_ref[:, :_HH] = a0_ref[...] + b[:, :_HH]
    o_ref[:, _HH:] = a1_ref[...] + b[:, _HH:]


_fin = pl.pallas_call(
    _fin_body,
    grid=(_N // _RB,),
    in_specs=[
        pl.BlockSpec((_RB, _HH), lambda i: (i, 0)),
        pl.BlockSpec((_RB, _HH), lambda i: (i, 0)),
        pl.BlockSpec((1, _H), lambda i: (0, 0)),
    ],
    out_specs=pl.BlockSpec((_RB, _H), lambda i: (i, 0)),
    out_shape=jax.ShapeDtypeStruct((_N, _H), jnp.float32),
)


def kernel(x, edge_index, edge_weight, batch,
           W0, b0, W1, b1, W2, b2, W3, b3, W4, b4):
    del batch
    loop = jnp.arange(_N, dtype=jnp.int32)
    padi = jnp.zeros((_EP - _ET,), jnp.int32)
    src_f = jnp.concatenate([edge_index[0], loop, padi])
    dst_f = jnp.concatenate([edge_index[1], loop, padi])
    w_p = jnp.concatenate([
        edge_weight.astype(jnp.float32),
        jnp.ones((_N,), jnp.float32),
        jnp.zeros((_EP - _ET,), jnp.float32),
    ]).reshape(_NW, 1, _EPW)
    src_p = src_f.reshape(_NW, 1, _EPW)
    dst_p = dst_f.reshape(_NW, 1, _EPW)

    _deg_partials, _edge_norm, _agg = _sc_kernels()
    partials = _deg_partials(dst_p, w_p)
    dis = _dis_tc(partials.reshape(_NW, _N)).reshape(_N)
    nrm = _edge_norm(dis, src_p, dst_p, w_p)

    t0, t1 = _mm0(x, W0)
    a0, a1 = _agg(t0, t1, src_p, dst_p, nrm)
    for (bb, WW) in ((b0, W1), (b1, W2), (b2, W3), (b3, W4)):
        t0, t1 = _cmm(a0, a1, bb.reshape(1, _H), WW)
        a0, a1 = _agg(t0, t1, src_p, dst_p, nrm)
    return _fin(a0, a1, b4.reshape(1, _H))


# speedup vs baseline: 1.2682x; 1.0057x over previous
import functools

import jax
import jax.numpy as jnp
from jax import lax
from jax.experimental import pallas as pl
from jax.experimental.pallas import tpu as pltpu
from jax.experimental.pallas import tpu_sc as plsc

_N = 10000
_E = 160000
_D = 256
_H = 256
_HH = _H // 2

_NC = 2
_NS = 16
_L = 16
_NW = _NC * _NS

_ET = _E + _N
_GP = -(-_ET // (_NW * _L))
_EPW = _GP * _L
_EP = _NW * _EPW

_GE = 32
_EPT = 2 * _EPW
_GT = _EPT // _GE

_RQ = 624
_RQL = _N - _RQ * (_NS - 1)


def _deg_partials_body(dst_hbm, w_hbm, out_hbm, dstv, wv, deg):
    c = lax.axis_index("c")
    s = lax.axis_index("s")
    wid = s * _NC + c
    pltpu.sync_copy(dst_hbm.at[wid, 0], dstv)
    pltpu.sync_copy(w_hbm.at[wid, 0], wv)
    zero = jnp.zeros((_L,), jnp.float32)

    def _zero(i, _):
        deg[pl.ds(i * _L, _L)] = zero
        return 0

    lax.fori_loop(0, _N // _L, _zero, 0)

    def _acc(i, _):
        d16 = dstv[pl.ds(i * _L, _L)]
        w16 = wv[pl.ds(i * _L, _L)]
        plsc.addupdate_scatter(deg, [d16], w16)
        return 0

    lax.fori_loop(0, _GP, _acc, 0)
    pltpu.sync_copy(deg, out_hbm.at[wid, 0])


def _dis_body(p_ref, dis_ref):
    deg = jnp.sum(p_ref[...], axis=0, keepdims=True)
    dis_ref[...] = jnp.where(
        deg > 0.0, lax.rsqrt(jnp.maximum(deg, 1e-12)), 0.0)


_dis_tc = pl.pallas_call(
    _dis_body,
    out_shape=jax.ShapeDtypeStruct((1, _N), jnp.float32),
)


def _edge_norm_body(dis_hbm, src_hbm, dst_hbm, w_hbm, out_hbm,
                    disv, srcv, dstv, wv, nv):
    c = lax.axis_index("c")
    s = lax.axis_index("s")
    wid = s * _NC + c
    pltpu.sync_copy(dis_hbm, disv)
    pltpu.sync_copy(src_hbm.at[wid, 0], srcv)
    pltpu.sync_copy(dst_hbm.at[wid, 0], dstv)
    pltpu.sync_copy(w_hbm.at[wid, 0], wv)

    def _grp(i, _):
        sl = pl.ds(i * _L, _L)
        a = plsc.load_gather(disv, [srcv[sl]])
        b = plsc.load_gather(disv, [dstv[sl]])
        nv[sl] = a * wv[sl] * b
        return 0

    lax.fori_loop(0, _GP, _grp, 0)
    pltpu.sync_copy(nv, out_hbm.at[wid, 0])


_NB = 4
_PF = 2
_NG = 2 * _GP
_NGM = (_NG // _NB) * _NB


def _agg_body(t0_hbm, t1_hbm, src_hbm, dst_hbm, nrm_hbm, agg0_hbm, agg1_hbm,
              srcv, dstv, nrmv, rows, acc, gsem, ssem):
    c = lax.axis_index("c")
    s = lax.axis_index("s")

    z16 = jnp.zeros((_L,), jnp.float32)
    for i in range(_L):
        for k in range(_HH // _L):
            rows[0, i, pl.ds(k * _L, _L)] = z16
    r0 = s * _RQ
    for q in range(_RQL // _L):
        pltpu.sync_copy(rows.at[0], acc.at[pl.ds(r0 + q * _L, _L)])
    plsc.subcore_barrier()

    pltpu.sync_copy(src_hbm.at[s * 2, 0], srcv.at[pl.ds(0, _EPW)])
    pltpu.sync_copy(src_hbm.at[s * 2 + 1, 0], srcv.at[pl.ds(_EPW, _EPW)])
    pltpu.sync_copy(dst_hbm.at[s * 2, 0], dstv.at[pl.ds(0, _EPW)])
    pltpu.sync_copy(dst_hbm.at[s * 2 + 1, 0], dstv.at[pl.ds(_EPW, _EPW)])
    pltpu.sync_copy(nrm_hbm.at[s * 2, 0], nrmv.at[pl.ds(0, _EPW)])
    pltpu.sync_copy(nrm_hbm.at[s * 2 + 1, 0], nrmv.at[pl.ds(_EPW, _EPW)])

    t_hbm = t0_hbm

    def _gather(g, j):
        s16 = srcv[pl.ds(g * _L, _L)]

        @pl.when(c == 0)
        def _():
            pltpu.async_copy(t0_hbm.at[s16], rows.at[j], gsem)

        @pl.when(c == 1)
        def _():
            pltpu.async_copy(t1_hbm.at[s16], rows.at[j], gsem)

    def _wait_gather(j):
        z = jnp.zeros((_L,), jnp.int32)
        pltpu.make_async_copy(t_hbm.at[z], rows.at[j], gsem).wait()

    def _scale_scatter(g, j):
        n16 = nrmv[pl.ds(g * _L, _L)]
        dnums = lax.GatherDimensionNumbers(
            offset_dims=(), collapsed_slice_dims=(0,), start_index_map=(0,))
        for i2 in range(_L):
            idx = jnp.full((_L, 1), i2, jnp.int32)
            spl = lax.gather(n16, idx, dnums, (1,),
                             mode=lax.GatherScatterMode.PROMISE_IN_BOUNDS)
            for k in range(_HH // _L):
                ksl = pl.ds(k * _L, _L)
                rows[j, i2, ksl] = rows[j, i2, ksl] * spl
        d16 = dstv[pl.ds(g * _L, _L)]
        pltpu.async_copy(rows.at[j], acc.at[d16], ssem, add=True)

    def _drain_scatter(j):
        z = jnp.zeros((_L,), jnp.int32)
        pltpu.make_async_copy(rows.at[j], acc.at[z], ssem).wait()

    for j in range(_PF):
        _gather(j, j)

    def _chunk(m, _):
        g0 = m * _NB
        for jj in range(_NB):
            g = g0 + jj

            @pl.when(g >= _PF)
            def _():
                _drain_scatter((jj + _PF) % _NB)

            _gather(g + _PF, (jj + _PF) % _NB)
            _wait_gather(jj)
            _scale_scatter(g, jj)
        return 0

    lax.fori_loop(0, _NGM // _NB, _chunk, 0)

    for g in range(_NGM, _NG):
        jj = g % _NB
        _wait_gather(jj)
        _scale_scatter(g, jj)
    for jj in range(_PF + (_NG - _NGM)):
        _drain_scatter(jj)

    plsc.subcore_barrier()

    @pl.when(c == 0)
    def _():
        pltpu.sync_copy(acc.at[pl.ds(s * _RQ, _RQ)],
                        agg0_hbm.at[pl.ds(s * _RQ, _RQ)])

    @pl.when(c == 1)
    def _():
        pltpu.sync_copy(acc.at[pl.ds(s * _RQ, _RQ)],
                        agg1_hbm.at[pl.ds(s * _RQ, _RQ)])

    tail = _RQ * (_NS - 1)

    @pl.when(jnp.logical_and(c == 0, s == _NS - 1))
    def _():
        pltpu.sync_copy(acc.at[pl.ds(tail + _RQ, _RQL - _RQ)],
                        agg0_hbm.at[pl.ds(tail + _RQ, _RQL - _RQ)])

    @pl.when(jnp.logical_and(c == 1, s == _NS - 1))
    def _():
        pltpu.sync_copy(acc.at[pl.ds(tail + _RQ, _RQL - _RQ)],
                        agg1_hbm.at[pl.ds(tail + _RQ, _RQL - _RQ)])


@functools.cache
def _sc_kernels():
    mesh = plsc.VectorSubcoreMesh(core_axis_name="c", subcore_axis_name="s",
                                  num_cores=_NC, num_subcores=_NS)
    params = pltpu.CompilerParams(needs_layout_passes=False)
    deg = pl.kernel(
        _deg_partials_body,
        out_type=jax.ShapeDtypeStruct((_NW, 1, _N), jnp.float32),
        mesh=mesh,
        compiler_params=params,
        scratch_types=[
            pltpu.VMEM((_EPW,), jnp.int32),
            pltpu.VMEM((_EPW,), jnp.float32),
            pltpu.VMEM((_N,), jnp.float32),
        ],
    )
    enorm = pl.kernel(
        _edge_norm_body,
        out_type=jax.ShapeDtypeStruct((_NW, 1, _EPW), jnp.float32),
        mesh=mesh,
        compiler_params=params,
        scratch_types=[
            pltpu.VMEM((_N,), jnp.float32),
            pltpu.VMEM((_EPW,), jnp.int32),
            pltpu.VMEM((_EPW,), jnp.int32),
            pltpu.VMEM((_EPW,), jnp.float32),
            pltpu.VMEM((_EPW,), jnp.float32),
        ],
    )
    agg = pl.kernel(
        _agg_body,
        out_type=[jax.ShapeDtypeStruct((_N, _HH), jnp.float32)] * 2,
        mesh=mesh,
        compiler_params=params,
        scratch_types=[
            pltpu.VMEM((2 * _EPW,), jnp.int32),
            pltpu.VMEM((2 * _EPW,), jnp.int32),
            pltpu.VMEM((2 * _EPW,), jnp.float32),
            pltpu.VMEM((_NB, _L, _HH), jnp.float32),
            pltpu.VMEM_SHARED((_N, _HH), jnp.float32),
            pltpu.SemaphoreType.DMA,
            pltpu.SemaphoreType.DMA,
        ],
    )
    return deg, enorm, agg


_RB = 2000


def _mm0_body(x_ref, w_ref, t0_ref, t1_ref):
    t = lax.dot_general(x_ref[...], w_ref[...], (((1,), (1,)), ((), ())),
                        preferred_element_type=jnp.float32)
    t0_ref[...] = t[:, :_HH]
    t1_ref[...] = t[:, _HH:]


_mm0 = pl.pallas_call(
    _mm0_body,
    grid=(_N // _RB,),
    in_specs=[
        pl.BlockSpec((_RB, _D), lambda i: (i, 0)),
        pl.BlockSpec((_H, _D), lambda i: (0, 0)),
    ],
    out_specs=[pl.BlockSpec((_RB, _HH), lambda i: (i, 0))] * 2,
    out_shape=[jax.ShapeDtypeStruct((_N, _HH), jnp.float32)] * 2,
)


def _cm_body(a0_ref, a1_ref, b_ref, w_ref, t0_ref, t1_ref):
    b = b_ref[...]
    h0 = jnp.maximum(a0_ref[...] + b[:, :_HH], 0.0)
    h1 = jnp.maximum(a1_ref[...] + b[:, _HH:], 0.0)
    w = w_ref[...]
    t = (lax.dot_general(h0, w[:, :_HH], (((1,), (1,)), ((), ())),
                         preferred_element_type=jnp.float32)
         + lax.dot_general(h1, w[:, _HH:], (((1,), (1,)), ((), ())),
                           preferred_element_type=jnp.float32))
    t0_ref[...] = t[:, :_HH]
    t1_ref[...] = t[:, _HH:]


_cmm = pl.pallas_call(
    _cm_body,
    grid=(_N // _RB,),
    in_specs=[
        pl.BlockSpec((_RB, _HH), lambda i: (i, 0)),
        pl.BlockSpec((_RB, _HH), lambda i: (i, 0)),
        pl.BlockSpec((1, _H), lambda i: (0, 0)),
        pl.BlockSpec((_H, _H), lambda i: (0, 0)),
    ],
    out_specs=[pl.BlockSpec((_RB, _HH), lambda i: (i, 0))] * 2,
    out_shape=[jax.ShapeDtypeStruct((_N, _HH), jnp.float32)] * 2,
)


def _fin_body(a0_ref, a1_ref, b_ref, o_ref):
    b = b_ref[...]
    o_ref[:, :_HH] = a0_ref[...] + b[:, :_HH]
    o_ref[:, _HH:] = a1_ref[...] + b[:, _HH:]


_fin = pl.pallas_call(
    _fin_body,
    grid=(_N // _RB,),
    in_specs=[
        pl.BlockSpec((_RB, _HH), lambda i: (i, 0)),
        pl.BlockSpec((_RB, _HH), lambda i: (i, 0)),
        pl.BlockSpec((1, _H), lambda i: (0, 0)),
    ],
    out_specs=pl.BlockSpec((_RB, _H), lambda i: (i, 0)),
    out_shape=jax.ShapeDtypeStruct((_N, _H), jnp.float32),
)


def kernel(x, edge_index, edge_weight, batch,
           W0, b0, W1, b1, W2, b2, W3, b3, W4, b4):
    del batch
    loop = jnp.arange(_N, dtype=jnp.int32)
    padi = jnp.zeros((_EP - _ET,), jnp.int32)
    src_f = jnp.concatenate([edge_index[0], loop, padi])
    dst_f = jnp.concatenate([edge_index[1], loop, padi])
    w_p = jnp.concatenate([
        edge_weight.astype(jnp.float32),
        jnp.ones((_N,), jnp.float32),
        jnp.zeros((_EP - _ET,), jnp.float32),
    ]).reshape(_NW, 1, _EPW)
    src_p = src_f.reshape(_NW, 1, _EPW)
    dst_p = dst_f.reshape(_NW, 1, _EPW)

    _deg_partials, _edge_norm, _agg = _sc_kernels()
    partials = _deg_partials(dst_p, w_p)
    dis = _dis_tc(partials.reshape(_NW, _N)).reshape(_N)
    nrm = _edge_norm(dis, src_p, dst_p, w_p)

    t0, t1 = _mm0(x, W0)
    a0, a1 = _agg(t0, t1, src_p, dst_p, nrm)
    for (bb, WW) in ((b0, W1), (b1, W2), (b2, W3), (b3, W4)):
        t0, t1 = _cmm(a0, a1, bb.reshape(1, _H), WW)
        a0, a1 = _agg(t0, t1, src_p, dst_p, nrm)
    return _fin(a0, a1, b4.reshape(1, _H))
